# baseline (speedup 1.0000x reference)
import functools

import jax
import jax.numpy as jnp
from jax import lax
from jax.experimental import pallas as pl
from jax.experimental.pallas import tpu as pltpu
from jax.experimental.pallas import tpu_sc as plsc

VOCAB = 100000
EMBED = 64
BATCH = 4096
HIST = 50
B = BATCH * HIST

_info = plsc.get_sparse_core_info()
NC = _info.num_cores
NS = _info.num_subcores
NW = NC * NS
BPW = B // NW
CH = 800
NCHUNK = BPW // CH


@functools.partial(
    pl.kernel,
    mesh=plsc.VectorSubcoreMesh(core_axis_name="c", subcore_axis_name="s"),
    out_type=jax.ShapeDtypeStruct((B, EMBED), jnp.float32),
    scratch_types=[
        pltpu.VMEM((BPW,), jnp.int32),
        pltpu.VMEM((2, CH, EMBED), jnp.float32),
        pltpu.SemaphoreType.DMA,
        pltpu.SemaphoreType.DMA,
    ],
    compiler_params=pltpu.CompilerParams(use_tc_tiling_on_sc=False),
)
def _gather_kernel(x_hbm, table_hbm, out_hbm, idx_v, rows_v, gsem, ssem):
    wid = lax.axis_index("s") * NC + lax.axis_index("c")
    base = wid * BPW
    pltpu.sync_copy(x_hbm.at[pl.ds(base, BPW)], idx_v)

    def gather(c):
        return pltpu.async_copy(
            table_hbm.at[idx_v.at[pl.ds(c * CH, CH)]], rows_v.at[c % 2], gsem)

    def store(c):
        return pltpu.async_copy(
            rows_v.at[c % 2], out_hbm.at[pl.ds(base + c * CH, CH)], ssem)

    stores = [None] * NCHUNK
    g = gather(0)
    for c in range(NCHUNK):
        g.wait()
        stores[c] = store(c)
        if c + 1 < NCHUNK:
            if c >= 1:
                stores[c - 1].wait()
            g = gather(c + 1)
    stores[NCHUNK - 1].wait()
    if NCHUNK >= 2:
        stores[NCHUNK - 2].wait()


def kernel(x, table):
    flat = x.reshape(B)
    out = _gather_kernel(flat, table)
    return out.reshape(BATCH, HIST, EMBED)


# speedup vs baseline: 1.0014x; 1.0014x over previous
import functools

import jax
import jax.numpy as jnp
from jax import lax
from jax.experimental import pallas as pl
from jax.experimental.pallas import tpu as pltpu
from jax.experimental.pallas import tpu_sc as plsc

VOCAB = 100000
EMBED = 64
BATCH = 4096
HIST = 50
B = BATCH * HIST

_info = plsc.get_sparse_core_info()
NC = _info.num_cores
NS = _info.num_subcores
NW = NC * NS
BPW = B // NW
CH = 800
NCHUNK = BPW // CH


@functools.partial(
    pl.kernel,
    mesh=plsc.VectorSubcoreMesh(core_axis_name="c", subcore_axis_name="s"),
    out_type=jax.ShapeDtypeStruct((B // CH, CH, EMBED), jnp.float32),
    scratch_types=[
        pltpu.VMEM((BPW,), jnp.int32),
        pltpu.VMEM((2, CH, EMBED), jnp.float32),
        pltpu.SemaphoreType.DMA,
        pltpu.SemaphoreType.DMA,
    ],
    compiler_params=pltpu.CompilerParams(use_tc_tiling_on_sc=False),
)
def _gather_kernel(x_hbm, table_hbm, out_hbm, idx_v, rows_v, gsem, ssem):
    wid = lax.axis_index("s") * NC + lax.axis_index("c")
    base = wid * BPW
    pltpu.sync_copy(x_hbm.at[pl.ds(base, BPW)], idx_v)

    def gather(c):
        return pltpu.async_copy(
            table_hbm.at[idx_v.at[pl.ds(c * CH, CH)]], rows_v.at[c % 2], gsem)

    def store(c):
        return pltpu.async_copy(
            rows_v.at[c % 2], out_hbm.at[wid * NCHUNK + c], ssem)

    stores = [None] * NCHUNK
    g = gather(0)
    for c in range(NCHUNK):
        g.wait()
        stores[c] = store(c)
        if c + 1 < NCHUNK:
            if c >= 1:
                stores[c - 1].wait()
            g = gather(c + 1)
    stores[NCHUNK - 1].wait()
    if NCHUNK >= 2:
        stores[NCHUNK - 2].wait()


def kernel(x, table):
    flat = x.reshape(B)
    out = _gather_kernel(flat, table)
    return out.reshape(BATCH, HIST, EMBED)


assert B // CH == NW * NCHUNK
